# baseline (speedup 1.0000x reference)
import dataclasses
import functools

import jax
import jax.numpy as jnp
from jax.experimental import pallas as pl
from jax.experimental.pallas import tpu as pltpu
from jax.experimental.pallas import tpu_sc as plsc

B = 16384
F = 26
D = 32
DA = 40
L = 16
NW = 32
RW = B // NW
C = 32
K = RW // C
K2 = K // 2
IPC = C * F
W = 104
GPC = IPC // W
IDXROWS = B * F // W


def _fire_gathers(tab_hbm, emb_buf, idx_buf, sem):
    for g in range(GPC):
        pltpu.async_copy(
            tab_hbm.at[idx_buf.at[g]], emb_buf.at[pl.ds(g * W, W)], sem)


def _wait_gathers(tab_hbm, emb_buf, sem):
    for g in range(GPC):
        pltpu.make_async_copy(
            tab_hbm.at[pl.ds(0, W)], emb_buf.at[pl.ds(g * W, W)], sem).wait()


def _compute_chunk(emb_buf, bias_buf, u_buf, out_buf, k):
    @pl.loop(0, C)
    def _(c):
        base = c * F
        s0 = emb_buf[base, pl.ds(0, L)]
        s1 = emb_buf[base, pl.ds(L, L)]
        q0 = s0 * s0
        q1 = s1 * s1
        for f in range(1, F):
            v0 = emb_buf[base + f, pl.ds(0, L)]
            v1 = emb_buf[base + f, pl.ds(L, L)]
            s0 = s0 + v0
            s1 = s1 + v1
            q0 = q0 + v0 * v0
            q1 = q1 + v1 * v1
        u_buf[pl.ds(c * D, L)] = s0 * s0 - q0
        u_buf[pl.ds(c * D + L, L)] = s1 * s1 - q1

    lanes = jax.lax.iota(jnp.int32, L)
    rowsel_u = lanes * D
    rowsel_t = lanes * F
    col_lin = jnp.full((L,), D, jnp.int32)
    for t in range(C // L):
        acc = plsc.load_gather(u_buf, [rowsel_u + t * L * D])
        for d in range(1, D):
            acc = acc + plsc.load_gather(u_buf, [rowsel_u + (t * L * D + d)])
        lin = plsc.load_gather(emb_buf, [rowsel_t + t * L * F, col_lin])
        for f in range(1, F):
            lin = lin + plsc.load_gather(
                emb_buf, [rowsel_t + (t * L * F + f), col_lin])
        out = 0.5 * acc + lin + bias_buf[...]
        out = jnp.minimum(jnp.maximum(out, -2.0), 2.0)
        out_buf[pl.ds(k * C + t * L, L)] = out


def kernel(x, emb_w, lin_w, bias):
    idx = x.astype(jnp.int32).reshape(IDXROWS, W)
    tab = jnp.concatenate(
        [emb_w, lin_w, jnp.zeros((emb_w.shape[0], DA - D - 1), jnp.float32)],
        axis=1)
    bias16 = jnp.broadcast_to(bias, (L,))
    mesh = plsc.VectorSubcoreMesh(core_axis_name="core",
                                  subcore_axis_name="subcore")
    cp = pltpu.CompilerParams(use_tc_tiling_on_sc=False)
    if "needs_layout_passes" in pltpu.CompilerParams.__dataclass_fields__:
        cp = dataclasses.replace(cp, needs_layout_passes=False)

    @functools.partial(
        pl.kernel,
        out_type=jax.ShapeDtypeStruct((B,), jnp.float32),
        mesh=mesh,
        compiler_params=cp,
        scratch_types=[
            pltpu.VMEM((2, IPC, DA), jnp.float32),
            pltpu.VMEM((2, GPC, W), jnp.int32),
            pltpu.VMEM((L,), jnp.float32),
            pltpu.VMEM((C * D,), jnp.float32),
            pltpu.VMEM((RW,), jnp.float32),
            pltpu.SemaphoreType.DMA,
            pltpu.SemaphoreType.DMA,
        ],
    )
    def run(idx_hbm, tab_hbm, bias_hbm, out_hbm,
            emb_buf, idx_buf, bias_buf, u_buf, out_buf, sem_a, sem_b):
        wid = jax.lax.axis_index("core") * 16 + jax.lax.axis_index("subcore")
        pltpu.sync_copy(bias_hbm, bias_buf)
        row0 = wid * (K * GPC)

        pltpu.sync_copy(idx_hbm.at[pl.ds(row0, GPC)], idx_buf.at[0])
        _fire_gathers(tab_hbm, emb_buf.at[0], idx_buf.at[0], sem_a)
        pltpu.sync_copy(idx_hbm.at[pl.ds(row0 + GPC, GPC)], idx_buf.at[1])

        @pl.loop(0, K2)
        def _(kk):
            ka = 2 * kk
            _fire_gathers(tab_hbm, emb_buf.at[1], idx_buf.at[1], sem_b)
            _wait_gathers(tab_hbm, emb_buf.at[0], sem_a)
            _compute_chunk(emb_buf.at[0], bias_buf, u_buf, out_buf, ka)

            @pl.when(kk < K2 - 1)
            def _():
                pltpu.sync_copy(
                    idx_hbm.at[pl.ds(row0 + (ka + 2) * GPC, GPC)],
                    idx_buf.at[0])
                _fire_gathers(tab_hbm, emb_buf.at[0], idx_buf.at[0], sem_a)

            _wait_gathers(tab_hbm, emb_buf.at[1], sem_b)
            _compute_chunk(emb_buf.at[1], bias_buf, u_buf, out_buf, ka + 1)

            @pl.when(kk < K2 - 1)
            def _():
                pltpu.sync_copy(
                    idx_hbm.at[pl.ds(row0 + (ka + 3) * GPC, GPC)],
                    idx_buf.at[1])

        pltpu.sync_copy(out_buf, out_hbm.at[pl.ds(wid * RW, RW)])

    out = run(idx, tab, bias16)
    return out.reshape(B)


# speedup vs baseline: 1.5452x; 1.5452x over previous
import dataclasses
import functools

import jax
import jax.numpy as jnp
from jax.experimental import pallas as pl
from jax.experimental.pallas import tpu as pltpu
from jax.experimental.pallas import tpu_sc as plsc

B = 16384
F = 26
D = 32
L = 16
NW = 32
RW = B // NW
C = 64
K = RW // C
IPC = C * F
W = 128
GPC = IPC // W
IDXROWS = B * F // W


def _fire_gathers(emb_hbm, lin_hbm, emb_buf, lin_buf, idx_buf, sem, base):
    cps = []
    for g in range(GPC):
        cps.append(pltpu.async_copy(
            emb_hbm.at[idx_buf.at[g]], emb_buf.at[pl.ds(g * W, W)], sem))
        cps.append(pltpu.async_copy(
            lin_hbm.at[idx_buf.at[g]], lin_buf.at[pl.ds(g * W, W)], sem))
    return cps


def _compute_chunk(emb_buf, lin_buf, bias_buf, u_buf, out_buf, k):
    @pl.loop(0, C)
    def _(c):
        base = c * F
        sa0, sa1 = plsc.unpack(emb_buf[base, pl.ds(0, D)],
                               format=plsc.PackFormat.INTERLEAVED)
        qa0 = sa0 * sa0
        qa1 = sa1 * sa1
        sb0, sb1 = plsc.unpack(emb_buf[base + 1, pl.ds(0, D)],
                               format=plsc.PackFormat.INTERLEAVED)
        qb0 = sb0 * sb0
        qb1 = sb1 * sb1
        for f in range(2, F, 2):
            v0, v1 = plsc.unpack(emb_buf[base + f, pl.ds(0, D)],
                                 format=plsc.PackFormat.INTERLEAVED)
            sa0 = sa0 + v0
            sa1 = sa1 + v1
            qa0 = qa0 + v0 * v0
            qa1 = qa1 + v1 * v1
        for f in range(3, F, 2):
            v0, v1 = plsc.unpack(emb_buf[base + f, pl.ds(0, D)],
                                 format=plsc.PackFormat.INTERLEAVED)
            sb0 = sb0 + v0
            sb1 = sb1 + v1
            qb0 = qb0 + v0 * v0
            qb1 = qb1 + v1 * v1
        s0 = sa0 + sb0
        s1 = sa1 + sb1
        u_buf[pl.ds(c * D, L)] = s0 * s0 - qa0 - qb0
        u_buf[pl.ds(c * D + L, L)] = s1 * s1 - qa1 - qb1

    lanes = jax.lax.iota(jnp.int32, L)
    rowsel_u = lanes * D
    rowsel_l = lanes * F
    for t in range(C // L):
        acc = plsc.load_gather(u_buf, [rowsel_u + t * L * D])
        for d in range(1, D):
            acc = acc + plsc.load_gather(u_buf, [rowsel_u + (t * L * D + d)])
        lin = plsc.load_gather(lin_buf, [rowsel_l + t * L * F])
        for f in range(1, F):
            lin = lin + plsc.load_gather(lin_buf, [rowsel_l + (t * L * F + f)])
        out = 0.5 * acc + lin + bias_buf[...]
        out = jnp.minimum(jnp.maximum(out, -2.0), 2.0)
        out_buf[pl.ds(k * C + t * L, L)] = out


def kernel(x, emb_w, lin_w, bias):
    idx = x.astype(jnp.int32).reshape(IDXROWS, W)
    emb16 = emb_w.astype(jnp.bfloat16)
    lin_flat = lin_w.reshape(-1)
    bias16 = jnp.broadcast_to(bias, (L,))
    mesh = plsc.VectorSubcoreMesh(core_axis_name="core",
                                  subcore_axis_name="subcore")
    cp = pltpu.CompilerParams(use_tc_tiling_on_sc=False)
    if "needs_layout_passes" in pltpu.CompilerParams.__dataclass_fields__:
        cp = dataclasses.replace(cp, needs_layout_passes=False)

    @functools.partial(
        pl.kernel,
        out_type=jax.ShapeDtypeStruct((B,), jnp.float32),
        mesh=mesh,
        compiler_params=cp,
        scratch_types=[
            pltpu.VMEM((2, IPC, D), jnp.bfloat16),
            pltpu.VMEM((2, IPC), jnp.float32),
            pltpu.VMEM((2, GPC, W), jnp.int32),
            pltpu.VMEM((L,), jnp.float32),
            pltpu.VMEM((C * D,), jnp.float32),
            pltpu.VMEM((RW,), jnp.float32),
            pltpu.SemaphoreType.DMA,
            pltpu.SemaphoreType.DMA,
        ],
    )
    def run(idx_hbm, emb_hbm, lin_hbm, bias_hbm, out_hbm,
            emb_buf, lin_buf, idx_buf, bias_buf, u_buf, out_buf,
            sem_a, sem_b):
        wid = jax.lax.axis_index("core") * 16 + jax.lax.axis_index("subcore")
        pltpu.sync_copy(bias_hbm, bias_buf)
        row0 = wid * (K * GPC)
        sems = (sem_a, sem_b)

        pltpu.sync_copy(idx_hbm.at[pl.ds(row0, GPC)], idx_buf.at[0])
        pend = _fire_gathers(emb_hbm, lin_hbm, emb_buf.at[0], lin_buf.at[0],
                             idx_buf.at[0], sems[0], 0)
        pltpu.sync_copy(idx_hbm.at[pl.ds(row0 + GPC, GPC)], idx_buf.at[1])

        for k in range(K):
            buf = k % 2
            nxt = 1 - buf
            if k + 1 < K:
                nxt_pend = _fire_gathers(
                    emb_hbm, lin_hbm, emb_buf.at[nxt], lin_buf.at[nxt],
                    idx_buf.at[nxt], sems[nxt], k + 1)
            for cp_ in pend:
                cp_.wait()
            _compute_chunk(emb_buf.at[buf], lin_buf.at[buf], bias_buf,
                           u_buf, out_buf, k)
            if k + 2 < K:
                pltpu.sync_copy(
                    idx_hbm.at[pl.ds(row0 + (k + 2) * GPC, GPC)],
                    idx_buf.at[buf])
            if k + 1 < K:
                pend = nxt_pend

        pltpu.sync_copy(out_buf, out_hbm.at[pl.ds(wid * RW, RW)])

    out = run(idx, emb16, lin_flat, bias16)
    return out.reshape(B)


# speedup vs baseline: 2.2462x; 1.4537x over previous
import dataclasses
import functools

import jax
import jax.numpy as jnp
from jax.experimental import pallas as pl
from jax.experimental.pallas import tpu as pltpu
from jax.experimental.pallas import tpu_sc as plsc

B = 16384
F = 26
D = 32
L = 16
NW = 32
RW = B // NW
C = 64
K = RW // C
IPC = C * F
W = 128
GPC = IPC // W
IDXROWS = B * F // W


def _fire_gathers(emb_hbm, lin_hbm, emb_buf, lin_buf, idx_buf, sem, base):
    cps = []
    for g in range(GPC):
        cps.append(pltpu.async_copy(
            emb_hbm.at[idx_buf.at[g]], emb_buf.at[pl.ds(g * W, W)], sem))
        cps.append(pltpu.async_copy(
            lin_hbm.at[idx_buf.at[g]], lin_buf.at[pl.ds(g * W, W)], sem))
    return cps


def _compute_chunk(emb_buf, lin_buf, bias_buf, u_buf, out_buf, k):
    @pl.loop(0, C)
    def _(c):
        base = c * F
        sa0 = emb_buf[base, pl.ds(0, L)]
        sa1 = emb_buf[base, pl.ds(L, L)]
        qa0 = sa0 * sa0
        qa1 = sa1 * sa1
        sb0 = emb_buf[base + 1, pl.ds(0, L)]
        sb1 = emb_buf[base + 1, pl.ds(L, L)]
        qb0 = sb0 * sb0
        qb1 = sb1 * sb1
        for f in range(2, F, 2):
            v0 = emb_buf[base + f, pl.ds(0, L)]
            v1 = emb_buf[base + f, pl.ds(L, L)]
            sa0 = sa0 + v0
            sa1 = sa1 + v1
            qa0 = qa0 + v0 * v0
            qa1 = qa1 + v1 * v1
        for f in range(3, F, 2):
            v0 = emb_buf[base + f, pl.ds(0, L)]
            v1 = emb_buf[base + f, pl.ds(L, L)]
            sb0 = sb0 + v0
            sb1 = sb1 + v1
            qb0 = qb0 + v0 * v0
            qb1 = qb1 + v1 * v1
        s0 = sa0 + sb0
        s1 = sa1 + sb1
        u_buf[pl.ds(c * D, L)] = s0 * s0 - qa0 - qb0
        u_buf[pl.ds(c * D + L, L)] = s1 * s1 - qa1 - qb1

    lanes = jax.lax.iota(jnp.int32, L)
    rowsel_u = lanes * D
    rowsel_l = lanes * F
    for t in range(C // L):
        acc = plsc.load_gather(u_buf, [rowsel_u + t * L * D])
        for d in range(1, D):
            acc = acc + plsc.load_gather(u_buf, [rowsel_u + (t * L * D + d)])
        lin = plsc.load_gather(lin_buf, [rowsel_l + t * L * F])
        for f in range(1, F):
            lin = lin + plsc.load_gather(lin_buf, [rowsel_l + (t * L * F + f)])
        out = 0.5 * acc + lin + bias_buf[...]
        out = jnp.minimum(jnp.maximum(out, -2.0), 2.0)
        out_buf[pl.ds(k * C + t * L, L)] = out


def kernel(x, emb_w, lin_w, bias):
    idx = x.astype(jnp.int32).reshape(IDXROWS, W)
    lin_flat = lin_w.reshape(-1)
    bias16 = jnp.broadcast_to(bias, (L,))
    mesh = plsc.VectorSubcoreMesh(core_axis_name="core",
                                  subcore_axis_name="subcore")
    cp = pltpu.CompilerParams(use_tc_tiling_on_sc=False)
    if "needs_layout_passes" in pltpu.CompilerParams.__dataclass_fields__:
        cp = dataclasses.replace(cp, needs_layout_passes=False)

    @functools.partial(
        pl.kernel,
        out_type=jax.ShapeDtypeStruct((B,), jnp.float32),
        mesh=mesh,
        compiler_params=cp,
        scratch_types=[
            pltpu.VMEM((2, IPC, D), jnp.float32),
            pltpu.VMEM((2, IPC), jnp.float32),
            pltpu.VMEM((2, GPC, W), jnp.int32),
            pltpu.VMEM((L,), jnp.float32),
            pltpu.VMEM((C * D,), jnp.float32),
            pltpu.VMEM((RW,), jnp.float32),
            pltpu.SemaphoreType.DMA,
            pltpu.SemaphoreType.DMA,
        ],
    )
    def run(idx_hbm, emb_hbm, lin_hbm, bias_hbm, out_hbm,
            emb_buf, lin_buf, idx_buf, bias_buf, u_buf, out_buf,
            sem_a, sem_b):
        wid = jax.lax.axis_index("core") * 16 + jax.lax.axis_index("subcore")
        pltpu.sync_copy(bias_hbm, bias_buf)
        row0 = wid * (K * GPC)
        sems = (sem_a, sem_b)

        pltpu.sync_copy(idx_hbm.at[pl.ds(row0, GPC)], idx_buf.at[0])
        pend = _fire_gathers(emb_hbm, lin_hbm, emb_buf.at[0], lin_buf.at[0],
                             idx_buf.at[0], sems[0], 0)
        pltpu.sync_copy(idx_hbm.at[pl.ds(row0 + GPC, GPC)], idx_buf.at[1])

        for k in range(K):
            buf = k % 2
            nxt = 1 - buf
            if k + 1 < K:
                nxt_pend = _fire_gathers(
                    emb_hbm, lin_hbm, emb_buf.at[nxt], lin_buf.at[nxt],
                    idx_buf.at[nxt], sems[nxt], k + 1)
            for cp_ in pend:
                cp_.wait()
            _compute_chunk(emb_buf.at[buf], lin_buf.at[buf], bias_buf,
                           u_buf, out_buf, k)
            if k + 2 < K:
                pltpu.sync_copy(
                    idx_hbm.at[pl.ds(row0 + (k + 2) * GPC, GPC)],
                    idx_buf.at[buf])
            if k + 1 < K:
                pend = nxt_pend

        pltpu.sync_copy(out_buf, out_hbm.at[pl.ds(wid * RW, RW)])

    out = run(idx, emb_w, lin_flat, bias16)
    return out.reshape(B)
